# baseline (speedup 1.0000x reference)
import functools

import jax
import jax.numpy as jnp
from jax import lax
from jax.experimental import pallas as pl
from jax.experimental.pallas import tpu as pltpu
from jax.experimental.pallas import tpu_sc as plsc

N = 100000
V = 95
D = 512
NC = 2
NS = 16
NW = NC * NS
L = 16
C = 64
G = C // L
U = 8
NCHUNKS = -(-N // C)
CPW = NCHUNKS // NW
EXTRA = NCHUNKS - CPW * NW
MAXLOC = CPW + 1
IDXBUF = MAXLOC * C


def _sc_embed(x, table_flat):
    mesh = plsc.VectorSubcoreMesh(core_axis_name="c", subcore_axis_name="s")

    @functools.partial(
        pl.kernel,
        mesh=mesh,
        compiler_params=pltpu.CompilerParams(needs_layout_passes=False),
        out_type=jax.ShapeDtypeStruct((N * D,), jnp.float32),
        scratch_types=[
            pltpu.VMEM((V * D,), jnp.float32),
            pltpu.VMEM((IDXBUF,), jnp.int32),
            pltpu.VMEM((C * D,), jnp.float32),
            pltpu.VMEM((C * D,), jnp.float32),
            pltpu.SemaphoreType.DMA,
            pltpu.SemaphoreType.DMA,
        ],
    )
    def k(x_hbm, table_hbm, out_hbm, tab_v, idx_v, rows0, rows1, s0, s1):
        cid = lax.axis_index("c")
        sid = lax.axis_index("s")
        wid = sid * NC + cid
        nloc = CPW + jnp.where(wid < EXTRA, 1, 0)
        start = wid * CPW + jnp.minimum(wid, EXTRA)
        load_base = jnp.minimum(start * C, N - IDXBUF)

        rows = (rows0, rows1)
        ssem = (s0, s1)

        pltpu.sync_copy(table_hbm, tab_v)
        pltpu.sync_copy(x_hbm.at[pl.ds(load_base, IDXBUF)], idx_v)

        lanes = lax.iota(jnp.int32, L)

        def off_of(i):
            return jnp.minimum((start + i) * C, N - C)

        def fill(i, b):
            bo = off_of(i) - load_base
            for g in range(G):
                rvec = idx_v[pl.ds(bo + g * L, L)]
                src0 = rvec * D
                dst0 = (g * L + lanes) * D

                def col(t, carry):
                    s_, d_ = carry
                    for u in range(U):
                        v = plsc.load_gather(tab_v, [s_ + u])
                        plsc.store_scatter(rows[b], [d_ + u], v)
                    return (s_ + U, d_ + U)

                lax.fori_loop(0, D // U, col, (src0, dst0))

        def scatter(i, b):
            return pltpu.make_async_copy(
                rows[b], out_hbm.at[pl.ds(off_of(i) * D, C * D)], ssem[b])

        def body(j, _):
            for b in range(2):
                i = 2 * j + b

                @pl.when(i < nloc)
                def _():
                    @pl.when(i >= 2)
                    def _():
                        scatter(i, b).wait()
                    fill(i, b)
                    scatter(i, b).start()
            return 0

        lax.fori_loop(0, (MAXLOC + 1) // 2, body, 0)

        for b in range(2):
            @pl.when(nloc > b)
            def _():
                scatter(b, b).wait()

    return k(x, table_flat)


def kernel(x, table):
    out = _sc_embed(x.astype(jnp.int32), table.reshape(-1))
    return out.reshape(N, D)


# speedup vs baseline: 4.2972x; 4.2972x over previous
import functools

import jax
import jax.numpy as jnp
from jax import lax
from jax.experimental import pallas as pl
from jax.experimental.pallas import tpu as pltpu
from jax.experimental.pallas import tpu_sc as plsc

N = 100000
V = 95
D = 512
NC = 2
NS = 16
NW = NC * NS
L = 16
C = 64
G = C // L
U = 8
NCHUNKS = -(-N // C)
CPW = NCHUNKS // NW
EXTRA = NCHUNKS - CPW * NW
MAXLOC = CPW + 1
IDXBUF = MAXLOC * C


def _sc_embed(x, table_flat):
    mesh = plsc.VectorSubcoreMesh(core_axis_name="c", subcore_axis_name="s")

    @functools.partial(
        pl.kernel,
        mesh=mesh,
        compiler_params=pltpu.CompilerParams(needs_layout_passes=False),
        out_type=jax.ShapeDtypeStruct((N * D,), jnp.float32),
        scratch_types=[
            pltpu.VMEM((V * D,), jnp.float32),
            pltpu.VMEM((IDXBUF,), jnp.int32),
            pltpu.VMEM((C * D,), jnp.float32),
            pltpu.VMEM((C * D,), jnp.float32),
            pltpu.SemaphoreType.DMA,
            pltpu.SemaphoreType.DMA,
        ],
    )
    def k(x_hbm, table_hbm, out_hbm, tab_v, idx_v, rows0, rows1, s0, s1):
        cid = lax.axis_index("c")
        sid = lax.axis_index("s")
        wid = sid * NC + cid
        nloc = CPW + jnp.where(wid < EXTRA, 1, 0)
        start = wid * CPW + jnp.minimum(wid, EXTRA)
        load_base = jnp.minimum(start * C, N - IDXBUF)

        rows = (rows0, rows1)
        ssem = (s0, s1)

        pltpu.sync_copy(table_hbm, tab_v)
        pltpu.sync_copy(x_hbm.at[pl.ds(load_base, IDXBUF)], idx_v)

        lanes = lax.iota(jnp.int32, L)
        zeros = jnp.zeros((L,), jnp.int32)

        def off_of(i):
            return jnp.minimum((start + i) * C, N - C)

        def fill(i, b):
            bo = off_of(i) - load_base

            def rowpair(p, _):
                r0 = bo + 2 * p
                rs0 = plsc.load_gather(idx_v, [zeros + r0])
                rs1 = plsc.load_gather(idx_v, [zeros + (r0 + 1)])
                o0 = 2 * p * D

                def col(t, carry):
                    s0_, s1_ = carry
                    to = o0 + t * (U * L)
                    for u in range(U):
                        v0 = plsc.load_gather(tab_v, [s0_ + u * L])
                        v1 = plsc.load_gather(tab_v, [s1_ + u * L])
                        rows[b][pl.ds(to + u * L, L)] = v0
                        rows[b][pl.ds(to + D + u * L, L)] = v1
                    return (s0_ + U * L, s1_ + U * L)

                lax.fori_loop(0, D // (U * L), col,
                              (rs0 * D + lanes, rs1 * D + lanes))
                return 0

            lax.fori_loop(0, C // 2, rowpair, 0)

        def scatter(i, b):
            return pltpu.make_async_copy(
                rows[b], out_hbm.at[pl.ds(off_of(i) * D, C * D)], ssem[b])

        def body(j, _):
            for b in range(2):
                i = 2 * j + b

                @pl.when(i < nloc)
                def _():
                    @pl.when(i >= 2)
                    def _():
                        scatter(i, b).wait()
                    fill(i, b)
                    scatter(i, b).start()
            return 0

        lax.fori_loop(0, (MAXLOC + 1) // 2, body, 0)

        for b in range(2):
            @pl.when(nloc > b)
            def _():
                scatter(b, b).wait()

    return k(x, table_flat)


def kernel(x, table):
    out = _sc_embed(x.astype(jnp.int32), table.reshape(-1))
    return out.reshape(N, D)


# speedup vs baseline: 5.5890x; 1.3006x over previous
import functools

import jax
import jax.numpy as jnp
from jax import lax
from jax.experimental import pallas as pl
from jax.experimental.pallas import tpu as pltpu
from jax.experimental.pallas import tpu_sc as plsc

N = 100000
D = 512
NC = 2
NS = 16
NW = NC * NS
C = 56
K = 4
NCHUNKS = -(-N // C)
CPW = NCHUNKS // NW
EXTRA = NCHUNKS - CPW * NW
MAXLOC = CPW + 1
IDXBUF = MAXLOC * C


def _sc_gather(x, table):
    mesh = plsc.VectorSubcoreMesh(core_axis_name="c", subcore_axis_name="s")

    @functools.partial(
        pl.kernel,
        mesh=mesh,
        out_type=jax.ShapeDtypeStruct((N, D), jnp.float32),
        scratch_types=(
            [pltpu.VMEM((IDXBUF,), jnp.int32)]
            + [pltpu.VMEM((C, D), jnp.float32) for _ in range(K)]
            + [pltpu.SemaphoreType.DMA for _ in range(2 * K)]
        ),
    )
    def k(x_hbm, table_hbm, out_hbm, idx_v, *bufs):
        rows = bufs[:K]
        gsem = bufs[K:2 * K]
        ssem = bufs[2 * K:]
        cid = lax.axis_index("c")
        sid = lax.axis_index("s")
        wid = sid * NC + cid
        nloc = CPW + jnp.where(wid < EXTRA, 1, 0)
        start = wid * CPW + jnp.minimum(wid, EXTRA)
        load_base = jnp.minimum(start * C, N - IDXBUF)

        pltpu.sync_copy(x_hbm.at[pl.ds(load_base, IDXBUF)], idx_v)

        def off_of(i):
            return jnp.minimum((start + i) * C, N - C)

        def gather(i, b):
            bo = off_of(i) - load_base
            return pltpu.make_async_copy(
                table_hbm.at[idx_v.at[pl.ds(bo, C)]], rows[b], gsem[b])

        def scatter(i, b):
            return pltpu.make_async_copy(
                rows[b], out_hbm.at[pl.ds(off_of(i), C)], ssem[b])

        for b in range(K):
            @pl.when(b < nloc)
            def _():
                gather(b, b).start()

        def body(j, _):
            for b in range(K):
                i = K * j + b

                @pl.when(i < nloc)
                def _():
                    gather(i, b).wait()
                    scatter(i, b).start()

                    bp = (b + K - 1) % K

                    @pl.when(jnp.logical_and(i >= 1, i + K - 1 < nloc))
                    def _():
                        scatter(i - 1, bp).wait()
                        gather(i + K - 1, bp).start()
            return 0

        lax.fori_loop(0, (MAXLOC + K - 1) // K, body, 0)

        for b in range(K):
            @pl.when(nloc > b)
            def _():
                scatter(b, b).wait()

    return k(x, table)


def kernel(x, table):
    return _sc_gather(x.astype(jnp.int32), table)


# speedup vs baseline: 6.8859x; 1.2320x over previous
import functools

import jax
import jax.numpy as jnp
from jax import lax
from jax.experimental import pallas as pl
from jax.experimental.pallas import tpu as pltpu
from jax.experimental.pallas import tpu_sc as plsc

N = 100000
V = 95
D = 512
NC = 2
NS = 16
NW = NC * NS
L = 16
C = 64
G = C // L
U = 8
NCHUNKS = -(-N // C)
CPW = NCHUNKS // NW
EXTRA = NCHUNKS - CPW * NW
MAXLOC = CPW + 1
IDXBUF = MAXLOC * C


def _sc_embed(x, table_flat):
    mesh = plsc.VectorSubcoreMesh(core_axis_name="c", subcore_axis_name="s")

    @functools.partial(
        pl.kernel,
        mesh=mesh,
        compiler_params=pltpu.CompilerParams(needs_layout_passes=False),
        out_type=jax.ShapeDtypeStruct((N * D,), jnp.float32),
        scratch_types=[
            pltpu.VMEM((V * D,), jnp.float32),
            pltpu.VMEM((IDXBUF,), jnp.int32),
            pltpu.VMEM((C * D,), jnp.float32),
            pltpu.VMEM((C * D,), jnp.float32),
            pltpu.SemaphoreType.DMA,
            pltpu.SemaphoreType.DMA,
        ],
    )
    def k(x_hbm, table_hbm, out_hbm, tab_v, idx_v, rows0, rows1, s0, s1):
        cid = lax.axis_index("c")
        sid = lax.axis_index("s")
        wid = sid * NC + cid
        nloc = CPW + jnp.where(wid < EXTRA, 1, 0)
        start = wid * CPW + jnp.minimum(wid, EXTRA)
        load_base = jnp.minimum(start * C, N - IDXBUF)

        rows = (rows0, rows1)
        ssem = (s0, s1)

        pltpu.sync_copy(table_hbm, tab_v)
        pltpu.sync_copy(x_hbm.at[pl.ds(load_base, IDXBUF)], idx_v)

        lanes = lax.iota(jnp.int32, L)
        zeros = jnp.zeros((L,), jnp.int32)

        def off_of(i):
            return jnp.minimum((start + i) * C, N - C)

        def fill(i, b):
            bo = off_of(i) - load_base

            @plsc.parallel_loop(0, C, step=1, unroll=2)
            def _(r):
                rsplat = plsc.load_gather(idx_v, [zeros + (bo + r)])
                base = rsplat * D + lanes
                for t in range(D // L):
                    v = plsc.load_gather(tab_v, [base + t * L])
                    rows[b][pl.ds(r * D + t * L, L)] = v

        def scatter(i, b):
            return pltpu.make_async_copy(
                rows[b], out_hbm.at[pl.ds(off_of(i) * D, C * D)], ssem[b])

        def body(j, _):
            for b in range(2):
                i = 2 * j + b

                @pl.when(i < nloc)
                def _():
                    @pl.when(i >= 2)
                    def _():
                        scatter(i, b).wait()
                    fill(i, b)
                    scatter(i, b).start()
            return 0

        lax.fori_loop(0, (MAXLOC + 1) // 2, body, 0)

        for b in range(2):
            @pl.when(nloc > b)
            def _():
                scatter(b, b).wait()

    return k(x, table_flat)


def kernel(x, table):
    out = _sc_embed(x.astype(jnp.int32), table.reshape(-1))
    return out.reshape(N, D)
